# baseline (speedup 1.0000x reference)
import functools

import jax
import jax.numpy as jnp
from jax.experimental import pallas as pl
from jax.experimental.pallas import tpu as pltpu
from jax.experimental.pallas import tpu_sc as plsc

EPS = 1e-6
K = 16
SLOPE = 0.2

_F32 = jnp.float32


def _dot(a, b):
    return jax.lax.dot_general(a, b, (((1,), (0,)), ((), ())),
                               preferred_element_type=_F32,
                               precision=jax.lax.Precision.DEFAULT)


def _pad128(w):
    return ((w + 127) // 128) * 128


def _knn_body(pb_ref, pa_ref, idx_ref, *, n, rb, k):
    pb = pb_ref[0]
    pa = pa_ref[0]
    b = pl.program_id(0)
    ones = jnp.ones((rb, 8), _F32)
    sqb = jnp.sum(pb * pb, axis=1, keepdims=True)
    sqa = jax.lax.dot_general(
        ones, pa * pa, (((1,), (1,)), ((), ())), preferred_element_type=_F32,
        precision=jax.lax.Precision.HIGHEST)
    g = jax.lax.dot_general(
        pb, pa, (((1,), (1,)), ((), ())), preferred_element_type=_F32,
        precision=jax.lax.Precision.DEFAULT)
    d2 = sqb + sqa - 2.0 * g
    iota = jax.lax.broadcasted_iota(jnp.int32, (rb, n), 1)
    big = jnp.int32(2 ** 30)
    inf = jnp.float32(jnp.inf)
    cols = []
    cur = d2
    for _ in range(k):
        m = jnp.min(cur, axis=1, keepdims=True)
        cand = jnp.where(cur <= m, iota, big)
        j = jnp.min(cand, axis=1, keepdims=True)
        cols.append(j + b * n)
        cur = jnp.where(iota == j, inf, cur)
    idx_ref[0] = jnp.concatenate(cols, axis=1)


def _knn(pts_pad, B, N, RB=1024):
    body = functools.partial(_knn_body, n=N, rb=RB, k=K)
    return pl.pallas_call(
        body,
        grid=(B, N // RB),
        in_specs=[
            pl.BlockSpec((1, RB, 8), lambda b, r: (b, r, 0)),
            pl.BlockSpec((1, N, 8), lambda b, r: (b, 0, 0)),
        ],
        out_specs=pl.BlockSpec((1, RB, K), lambda b, r: (b, r, 0)),
        out_shape=jax.ShapeDtypeStruct((B, N, K), jnp.int32),
    )(pts_pad, pts_pad)


def _sc_gather(table, idx2d, E, W, GW=None):
    if GW is None:
        GW = 256 if W <= 128 else 128
    mesh = plsc.VectorSubcoreMesh(core_axis_name="c", subcore_axis_name="s")

    @pl.kernel(out_type=jax.ShapeDtypeStruct((E, W), _F32), mesh=mesh)
    def gather_kernel(x_hbm, i_hbm, o_hbm):
        def body(i_vmem, o_vmem):
            pltpu.sync_copy(x_hbm.at[i_vmem.at[0]], o_vmem)

        pltpu.emit_pipeline(
            body,
            grid=(E // GW,),
            in_specs=[pl.BlockSpec((1, GW), index_map=lambda i: (0, i))],
            out_specs=[pl.BlockSpec((GW, W), index_map=lambda i: (i, 0))],
            core_axis_name=("c", "s"),
            dimension_semantics=(pltpu.PARALLEL,),
        )(i_hbm, o_hbm)

    return gather_kernel(table, idx2d)


def _edge_body(g_ref, x_ref, w_ref, o_ref, *, c, co, pb, wpad_out):
    c3 = 3 * c
    G = g_ref[...][:, :c3]
    X = x_ref[...][:, :c3]
    Xr = jnp.broadcast_to(X[:, None, :], (pb, K, c3)).reshape(K * pb, c3)
    diff = G - Xr
    W2 = w_ref[...]
    qm, dm = [], []
    for m in range(3):
        ym = jnp.concatenate(
            [diff[:, m * c:(m + 1) * c], Xr[:, m * c:(m + 1) * c]], axis=1)
        qd = _dot(ym, W2)
        qm.append(qd[:, :co])
        dm.append(qd[:, 128:128 + co])
    s = qm[0] * dm[0] + qm[1] * dm[1] + qm[2] * dm[2]
    n2 = dm[0] * dm[0] + dm[1] * dm[1] + dm[2] * dm[2]
    denom = jnp.sqrt(n2) + EPS
    coef = (SLOPE - 1.0) * jnp.minimum(s, 0.0) / (denom * denom)
    out = jnp.concatenate([qm[m] + coef * dm[m] for m in range(3)], axis=1)
    out = out.reshape(pb, K, 3 * co).sum(axis=1) * (1.0 / K)
    pad = wpad_out - 3 * co
    if pad:
        out = jnp.concatenate([out, jnp.zeros((pb, pad), _F32)], axis=1)
    o_ref[...] = out


def _edge(G, X, W2, c, co, PB=256):
    BN = X.shape[0]
    wpad_out = _pad128(3 * co)
    body = functools.partial(_edge_body, c=c, co=co, pb=PB, wpad_out=wpad_out)
    return pl.pallas_call(
        body,
        grid=(BN // PB,),
        in_specs=[
            pl.BlockSpec((K * PB, G.shape[1]), lambda r: (r, 0)),
            pl.BlockSpec((PB, X.shape[1]), lambda r: (r, 0)),
            pl.BlockSpec(W2.shape, lambda r: (0, 0)),
        ],
        out_specs=pl.BlockSpec((PB, wpad_out), lambda r: (r, 0)),
        out_shape=jax.ShapeDtypeStruct((BN, wpad_out), _F32),
    )(G, X, W2)


def _mslices(x, c):
    return [x[:, m * c:(m + 1) * c] for m in range(3)]


def _vec_lna_rows(q, d, c):
    qm, dm = _mslices(q, c), _mslices(d, c)
    s = qm[0] * dm[0] + qm[1] * dm[1] + qm[2] * dm[2]
    n2 = dm[0] * dm[0] + dm[1] * dm[1] + dm[2] * dm[2]
    denom = jnp.sqrt(n2) + EPS
    coef = (SLOPE - 1.0) * jnp.minimum(s, 0.0) / (denom * denom)
    return jnp.concatenate([qm[m] + coef * dm[m] for m in range(3)], axis=1)


def _tail_body(x1_ref, x2_ref, x3_ref, x4_ref,
               wqc_ref, wdc_ref, wf1q_ref, wf1d_ref, wf2q_ref, wf2d_ref,
               o_ref):
    x1, x2, x3, x4 = x1_ref[...], x2_ref[...], x3_ref[...], x4_ref[...]
    xs = [(x1, 16), (x2, 32), (x3, 64), (x4, 128)]
    xcat = jnp.concatenate(
        [x[:, m * c:(m + 1) * c] for m in range(3) for (x, c) in xs], axis=1)
    q = _dot(xcat, wqc_ref[...])
    d = _dot(xcat, wdc_ref[...])
    d0, d1, d2 = d[:, 0:1], d[:, 1:2], d[:, 2:3]
    n2 = d0 * d0 + d1 * d1 + d2 * d2
    denom = jnp.sqrt(n2) + EPS
    qm = _mslices(q, 128)
    s = qm[0] * d0 + qm[1] * d1 + qm[2] * d2
    coef = (SLOPE - 1.0) * jnp.minimum(s, 0.0) / (denom * denom)
    xc_m = [qm[0] + coef * d0, qm[1] + coef * d1, qm[2] + coef * d2]
    xc = jnp.concatenate(xc_m, axis=1)
    nrows = xc.shape[0]
    xm = jnp.broadcast_to(jnp.mean(xc, axis=0, keepdims=True), (nrows, 384))
    xm_m = _mslices(xm, 128)
    xcat2 = jnp.concatenate(
        [xc_m[0], xm_m[0], xc_m[1], xm_m[1], xc_m[2], xm_m[2]], axis=1)
    z = _vec_lna_rows(_dot(xcat2, wf1q_ref[...]), _dot(xcat2, wf1d_ref[...]), 128)
    x0 = _vec_lna_rows(_dot(z, wf2q_ref[...]), _dot(z, wf2d_ref[...]), 256)
    xc2m, x0m = _mslices(xcat2, 256), _mslices(x0, 256)
    o_ref[...] = xc2m[0] * x0m[0] + xc2m[1] * x0m[1] + xc2m[2] * x0m[2]


def _tail(x1, x2, x3, x4, wqc, wdc, wf1q, wf1d, wf2q, wf2d, B, N):
    full = lambda a: pl.BlockSpec(a.shape, lambda b: (0, 0))
    row = lambda a: pl.BlockSpec((N, a.shape[1]), lambda b: (b, 0))
    return pl.pallas_call(
        _tail_body,
        grid=(B,),
        in_specs=[row(x1), row(x2), row(x3), row(x4),
                  full(wqc), full(wdc), full(wf1q), full(wf1d),
                  full(wf2q), full(wf2d)],
        out_specs=pl.BlockSpec((N, 256), lambda b: (b, 0)),
        out_shape=jax.ShapeDtypeStruct((B * N, 256), _F32),
    )(x1, x2, x3, x4, wqc, wdc, wf1q, wf1d, wf2q, wf2d)


def _blk3(w):
    wt = w.T
    i, o = wt.shape
    z = jnp.zeros((i, o), _F32)
    return jnp.concatenate([
        jnp.concatenate([wt, z, z], axis=1),
        jnp.concatenate([z, wt, z], axis=1),
        jnp.concatenate([z, z, wt], axis=1),
    ], axis=0)


def kernel(x, Wq1, Wd1, Wq2, Wd2, Wq3, Wd3, Wq4, Wd4, Wqc, Wdc, Wf1q, Wf1d, Wf2q, Wf2d):
    B, _, N = x.shape
    BN, E = B * N, B * N * K
    pts = jnp.transpose(x, (0, 2, 1))
    pts_pad = jnp.concatenate(
        [pts, jnp.zeros((B, N, 5), _F32)], axis=2)
    idx = _knn(pts_pad, B, N)
    idx2d = idx.reshape(1, E)

    cur = jnp.concatenate(
        [pts.reshape(BN, 3), jnp.zeros((BN, 125), _F32)], axis=1)
    layers = [(Wq1, Wd1, 1, 16), (Wq2, Wd2, 16, 32),
              (Wq3, Wd3, 32, 64), (Wq4, Wd4, 64, 128)]
    outs = []
    for (Wq, Wd, cin, cout) in layers:
        zpad = jnp.zeros((2 * cin, 128 - cout), _F32)
        W2 = jnp.concatenate([Wq.T, zpad, Wd.T, zpad], axis=1)
        nsplit = 4
        halves = []
        for h in range(nsplit):
            idx_h = jax.lax.slice(
                idx2d, (0, h * E // nsplit), (1, (h + 1) * E // nsplit))
            cur_h = jax.lax.slice(
                cur, (h * BN // nsplit, 0), ((h + 1) * BN // nsplit, cur.shape[1]))
            Gh = _sc_gather(cur, idx_h, E // nsplit, cur.shape[1])
            halves.append(_edge(Gh, cur_h, W2, cin, cout))
        cur = jnp.concatenate(halves, axis=0)
        outs.append(cur)

    inv_rows = _tail(outs[0], outs[1], outs[2], outs[3],
                     _blk3(Wqc), _blk3(Wdc), _blk3(Wf1q), _blk3(Wf1d),
                     _blk3(Wf2q), _blk3(Wf2d), B, N)
    return jnp.transpose(inv_rows.reshape(B, N, 256), (0, 2, 1))


# speedup vs baseline: 1.0287x; 1.0287x over previous
import functools

import jax
import jax.numpy as jnp
from jax.experimental import pallas as pl
from jax.experimental.pallas import tpu as pltpu
from jax.experimental.pallas import tpu_sc as plsc

EPS = 1e-6
K = 16
SLOPE = 0.2

_F32 = jnp.float32


def _dot(a, b):
    return jax.lax.dot_general(a, b, (((1,), (0,)), ((), ())),
                               preferred_element_type=_F32,
                               precision=jax.lax.Precision.DEFAULT)


def _pad128(w):
    return ((w + 127) // 128) * 128


def _knn_body(pb_ref, pa_ref, idx_ref, *, n, rb, k):
    pb = pb_ref[0]
    pa = pa_ref[0]
    b = pl.program_id(0)
    ones = jnp.ones((rb, 8), _F32)
    sqb = jnp.sum(pb * pb, axis=1, keepdims=True)
    sqa = jax.lax.dot_general(
        ones, pa * pa, (((1,), (1,)), ((), ())), preferred_element_type=_F32,
        precision=jax.lax.Precision.HIGHEST)
    g = jax.lax.dot_general(
        pb, pa, (((1,), (1,)), ((), ())), preferred_element_type=_F32,
        precision=jax.lax.Precision.DEFAULT)
    d2 = sqb + sqa - 2.0 * g
    iota = jax.lax.broadcasted_iota(jnp.int32, (rb, n), 1)
    big = jnp.int32(2 ** 30)
    inf = jnp.float32(jnp.inf)
    cols = []
    cur = d2
    for _ in range(k):
        m = jnp.min(cur, axis=1, keepdims=True)
        cand = jnp.where(cur <= m, iota, big)
        j = jnp.min(cand, axis=1, keepdims=True)
        cols.append(j + b * n)
        cur = jnp.where(iota == j, inf, cur)
    idx_ref[0] = jnp.concatenate(cols, axis=1)


def _knn(pts_pad, B, N, RB=1024):
    body = functools.partial(_knn_body, n=N, rb=RB, k=K)
    return pl.pallas_call(
        body,
        grid=(B, N // RB),
        in_specs=[
            pl.BlockSpec((1, RB, 8), lambda b, r: (b, r, 0)),
            pl.BlockSpec((1, N, 8), lambda b, r: (b, 0, 0)),
        ],
        out_specs=pl.BlockSpec((1, RB, K), lambda b, r: (b, r, 0)),
        out_shape=jax.ShapeDtypeStruct((B, N, K), jnp.int32),
    )(pts_pad, pts_pad)


def _sc_gather(table, idx2d, E, W, GW=None):
    if GW is None:
        GW = 256 if W <= 128 else 128
    mesh = plsc.VectorSubcoreMesh(core_axis_name="c", subcore_axis_name="s")

    @pl.kernel(out_type=jax.ShapeDtypeStruct((E, W), _F32), mesh=mesh)
    def gather_kernel(x_hbm, i_hbm, o_hbm):
        def body(i_vmem, o_vmem):
            pltpu.sync_copy(x_hbm.at[i_vmem.at[0]], o_vmem)

        pltpu.emit_pipeline(
            body,
            grid=(E // GW,),
            in_specs=[pl.BlockSpec((1, GW), index_map=lambda i: (0, i))],
            out_specs=[pl.BlockSpec((GW, W), index_map=lambda i: (i, 0))],
            core_axis_name=("c", "s"),
            dimension_semantics=(pltpu.PARALLEL,),
        )(i_hbm, o_hbm)

    return gather_kernel(table, idx2d)


def _edge_body(g_ref, x_ref, w_ref, o_ref, *, c, co, pb, wpad_out):
    c3 = 3 * c
    G = g_ref[...][:, :c3]
    X = x_ref[...][:, :c3]
    Xr = jnp.broadcast_to(X[:, None, :], (pb, K, c3)).reshape(K * pb, c3)
    diff = G - Xr
    W2 = w_ref[...]
    qm, dm = [], []
    for m in range(3):
        ym = jnp.concatenate(
            [diff[:, m * c:(m + 1) * c], Xr[:, m * c:(m + 1) * c]], axis=1)
        qd = _dot(ym, W2)
        qm.append(qd[:, :co])
        dm.append(qd[:, 128:128 + co])
    s = qm[0] * dm[0] + qm[1] * dm[1] + qm[2] * dm[2]
    n2 = dm[0] * dm[0] + dm[1] * dm[1] + dm[2] * dm[2]
    denom = jnp.sqrt(n2) + EPS
    coef = (SLOPE - 1.0) * jnp.minimum(s, 0.0) / (denom * denom)
    out = jnp.concatenate([qm[m] + coef * dm[m] for m in range(3)], axis=1)
    out = out.reshape(pb, K, 3 * co).sum(axis=1) * (1.0 / K)
    pad = wpad_out - 3 * co
    if pad:
        out = jnp.concatenate([out, jnp.zeros((pb, pad), _F32)], axis=1)
    o_ref[...] = out


def _edge(G, X, W2, c, co, PB=256):
    BN = X.shape[0]
    wpad_out = _pad128(3 * co)
    body = functools.partial(_edge_body, c=c, co=co, pb=PB, wpad_out=wpad_out)
    return pl.pallas_call(
        body,
        grid=(BN // PB,),
        in_specs=[
            pl.BlockSpec((K * PB, G.shape[1]), lambda r: (r, 0)),
            pl.BlockSpec((PB, X.shape[1]), lambda r: (r, 0)),
            pl.BlockSpec(W2.shape, lambda r: (0, 0)),
        ],
        out_specs=pl.BlockSpec((PB, wpad_out), lambda r: (r, 0)),
        out_shape=jax.ShapeDtypeStruct((BN, wpad_out), _F32),
    )(G, X, W2)


def _mslices(x, c):
    return [x[:, m * c:(m + 1) * c] for m in range(3)]


def _vec_lna_rows(q, d, c):
    qm, dm = _mslices(q, c), _mslices(d, c)
    s = qm[0] * dm[0] + qm[1] * dm[1] + qm[2] * dm[2]
    n2 = dm[0] * dm[0] + dm[1] * dm[1] + dm[2] * dm[2]
    denom = jnp.sqrt(n2) + EPS
    coef = (SLOPE - 1.0) * jnp.minimum(s, 0.0) / (denom * denom)
    return jnp.concatenate([qm[m] + coef * dm[m] for m in range(3)], axis=1)


def _tail_body(x1_ref, x2_ref, x3_ref, x4_ref,
               wqc_ref, wdc_ref, wf1q_ref, wf1d_ref, wf2q_ref, wf2d_ref,
               o_ref):
    x1, x2, x3, x4 = x1_ref[...], x2_ref[...], x3_ref[...], x4_ref[...]
    xs = [(x1, 16), (x2, 32), (x3, 64), (x4, 128)]
    xcat = jnp.concatenate(
        [x[:, m * c:(m + 1) * c] for m in range(3) for (x, c) in xs], axis=1)
    q = _dot(xcat, wqc_ref[...])
    d = _dot(xcat, wdc_ref[...])
    d0, d1, d2 = d[:, 0:1], d[:, 1:2], d[:, 2:3]
    n2 = d0 * d0 + d1 * d1 + d2 * d2
    denom = jnp.sqrt(n2) + EPS
    qm = _mslices(q, 128)
    s = qm[0] * d0 + qm[1] * d1 + qm[2] * d2
    coef = (SLOPE - 1.0) * jnp.minimum(s, 0.0) / (denom * denom)
    xc_m = [qm[0] + coef * d0, qm[1] + coef * d1, qm[2] + coef * d2]
    xc = jnp.concatenate(xc_m, axis=1)
    nrows = xc.shape[0]
    xm = jnp.broadcast_to(jnp.mean(xc, axis=0, keepdims=True), (nrows, 384))
    xm_m = _mslices(xm, 128)
    xcat2 = jnp.concatenate(
        [xc_m[0], xm_m[0], xc_m[1], xm_m[1], xc_m[2], xm_m[2]], axis=1)
    z = _vec_lna_rows(_dot(xcat2, wf1q_ref[...]), _dot(xcat2, wf1d_ref[...]), 128)
    x0 = _vec_lna_rows(_dot(z, wf2q_ref[...]), _dot(z, wf2d_ref[...]), 256)
    xc2m, x0m = _mslices(xcat2, 256), _mslices(x0, 256)
    o_ref[...] = xc2m[0] * x0m[0] + xc2m[1] * x0m[1] + xc2m[2] * x0m[2]


def _tail(x1, x2, x3, x4, wqc, wdc, wf1q, wf1d, wf2q, wf2d, B, N):
    full = lambda a: pl.BlockSpec(a.shape, lambda b: (0, 0))
    row = lambda a: pl.BlockSpec((N, a.shape[1]), lambda b: (b, 0))
    return pl.pallas_call(
        _tail_body,
        grid=(B,),
        in_specs=[row(x1), row(x2), row(x3), row(x4),
                  full(wqc), full(wdc), full(wf1q), full(wf1d),
                  full(wf2q), full(wf2d)],
        out_specs=pl.BlockSpec((N, 256), lambda b: (b, 0)),
        out_shape=jax.ShapeDtypeStruct((B * N, 256), _F32),
    )(x1, x2, x3, x4, wqc, wdc, wf1q, wf1d, wf2q, wf2d)


def _blk3(w):
    wt = w.T
    i, o = wt.shape
    z = jnp.zeros((i, o), _F32)
    return jnp.concatenate([
        jnp.concatenate([wt, z, z], axis=1),
        jnp.concatenate([z, wt, z], axis=1),
        jnp.concatenate([z, z, wt], axis=1),
    ], axis=0)


def kernel(x, Wq1, Wd1, Wq2, Wd2, Wq3, Wd3, Wq4, Wd4, Wqc, Wdc, Wf1q, Wf1d, Wf2q, Wf2d):
    B, _, N = x.shape
    BN, E = B * N, B * N * K
    pts = jnp.transpose(x, (0, 2, 1))
    pts_pad = jnp.concatenate(
        [pts, jnp.zeros((B, N, 5), _F32)], axis=2)
    idx = _knn(pts_pad, B, N)
    idx2d = idx.reshape(1, E)

    cur = jnp.concatenate(
        [pts.reshape(BN, 3), jnp.zeros((BN, 125), _F32)], axis=1)
    layers = [(Wq1, Wd1, 1, 16), (Wq2, Wd2, 16, 32),
              (Wq3, Wd3, 32, 64), (Wq4, Wd4, 64, 128)]
    outs = []
    for (Wq, Wd, cin, cout) in layers:
        zpad = jnp.zeros((2 * cin, 128 - cout), _F32)
        W2 = jnp.concatenate([Wq.T, zpad, Wd.T, zpad], axis=1)
        nsplit = 2
        halves = []
        for h in range(nsplit):
            idx_h = jax.lax.slice(
                idx2d, (0, h * E // nsplit), (1, (h + 1) * E // nsplit))
            cur_h = jax.lax.slice(
                cur, (h * BN // nsplit, 0), ((h + 1) * BN // nsplit, cur.shape[1]))
            Gh = _sc_gather(cur, idx_h, E // nsplit, cur.shape[1])
            halves.append(_edge(Gh, cur_h, W2, cin, cout))
        cur = jnp.concatenate(halves, axis=0)
        outs.append(cur)

    inv_rows = _tail(outs[0], outs[1], outs[2], outs[3],
                     _blk3(Wqc), _blk3(Wdc), _blk3(Wf1q), _blk3(Wf1d),
                     _blk3(Wf2q), _blk3(Wf2d), B, N)
    return jnp.transpose(inv_rows.reshape(B, N, 256), (0, 2, 1))
